# baseline (speedup 1.0000x reference)
import functools

import jax
import jax.numpy as jnp
from jax import lax
from jax.experimental import pallas as pl
from jax.experimental.pallas import tpu as pltpu
from jax.experimental.pallas import tpu_sc as plsc

N = 10000
E = 320000
H = 128
NC = 2
NS = 16
NW = NC * NS
EPW = E // NW
LANES = 16
BN = 2000
NB = N // BN
BE = 6400
GB = E // BE
UNROLL = 5
NITER = EPW // LANES


def _a_body(x_ref, wu_ref, a2_ref, g_ref, p1_ref, p2_ref):
    u = jnp.dot(x_ref[...], wu_ref[...], preferred_element_type=jnp.float32)
    hp = jnp.where(u > 0, u, 0.01 * u)
    g_ref[...] = jnp.where(u > 0, u, 0.0001 * u)
    p2v = lax.dot_general(a2_ref[...], hp, (((0,), (1,)), ((), ())),
                          preferred_element_type=jnp.float32)
    p1_ref[...] = p2v[0:1, :].reshape(1, 1, BN)
    p2_ref[...] = p2v[1:2, :].reshape(1, 1, BN)


def _a_call(x, W_u, A2):
    return pl.pallas_call(
        _a_body,
        grid=(NB,),
        in_specs=[
            pl.BlockSpec((BN, H), lambda i: (i, 0)),
            pl.BlockSpec((H, H), lambda i: (0, 0)),
            pl.BlockSpec((H, 2), lambda i: (0, 0)),
        ],
        out_specs=[
            pl.BlockSpec((BN, H), lambda i: (i, 0)),
            pl.BlockSpec((1, 1, BN), lambda i: (i, 0, 0)),
            pl.BlockSpec((1, 1, BN), lambda i: (i, 0, 0)),
        ],
        out_shape=[
            jax.ShapeDtypeStruct((N, H), jnp.float32),
            jax.ShapeDtypeStruct((NB, 1, BN), jnp.float32),
            jax.ShapeDtypeStruct((NB, 1, BN), jnp.float32),
        ],
    )(x, W_u, A2)


def _b_body(ea_ref, we_ref, wm_ref, t_ref, scr_ref, sem):
    i = pl.program_id(0)
    wem = jnp.dot(we_ref[...], wm_ref[...], preferred_element_type=jnp.float32)
    sT = lax.dot_general(wem, ea_ref[...], (((0,), (1,)), ((), ())),
                         preferred_element_type=jnp.float32)
    scr_ref[...] = sT
    cp = pltpu.make_async_copy(scr_ref, t_ref.at[pl.ds(i, 1), :], sem)
    cp.start()
    cp.wait()


def _b_call(edge_attr, W_e, W_m):
    d_edge = W_e.shape[0]
    return pl.pallas_call(
        _b_body,
        grid=(GB,),
        in_specs=[
            pl.BlockSpec((BE, d_edge), lambda i: (i, 0)),
            pl.BlockSpec((d_edge, H), lambda i: (0, 0)),
            pl.BlockSpec((H, 1), lambda i: (0, 0)),
        ],
        out_specs=pl.BlockSpec(memory_space=pltpu.MemorySpace.HBM),
        out_shape=jax.ShapeDtypeStruct((GB, BE), jnp.float32),
        scratch_shapes=[pltpu.VMEM((1, BE), jnp.float32),
                        pltpu.SemaphoreType.DMA],
    )(edge_attr, W_e, W_m)


def _sc_edge_body(ei_hbm, t_hbm, p1_hbm, p2_hbm, w_out, m_out,
                  src_v, dst_v, t_v, p1_v, p2_v, m_v, w_v, mrow_v):
    wid = lax.axis_index("s") * NC + lax.axis_index("c")
    base = wid * EPW
    pltpu.sync_copy(ei_hbm.at[pl.ds(base, EPW)], src_v)
    pltpu.sync_copy(ei_hbm.at[pl.ds(E + base, EPW)], dst_v)
    pltpu.sync_copy(t_hbm.at[pl.ds(base, EPW)], t_v)
    pltpu.sync_copy(p1_hbm, p1_v)
    pltpu.sync_copy(p2_hbm, p2_v)

    def body1(i, mx):
        for j in range(UNROLL):
            sl = pl.ds(pl.multiple_of((i * UNROLL + j) * LANES, LANES), LANES)
            si = src_v[sl]
            di = dst_v[sl]
            sc = plsc.load_gather(p1_v, [si]) + plsc.load_gather(p2_v, [di])
            e = jnp.exp(-jnp.abs(sc))
            sig = jnp.where(sc >= 0, 1.0 / (1.0 + e), e / (1.0 + e))
            m = sig * t_v[sl]
            m_v[sl] = m
            w_v[sl] = jnp.zeros((LANES,), jnp.float32)
            mx = jnp.maximum(mx, m)
        return mx

    mx0 = jnp.full((LANES,), -jnp.inf, dtype=jnp.float32)
    mx = lax.fori_loop(0, NITER // UNROLL, body1, mx0)
    mt = jnp.max(mx)

    def body2(i, carry):
        for j in range(UNROLL):
            sl = pl.ds(pl.multiple_of((i * UNROLL + j) * LANES, LANES), LANES)
            si = src_v[sl]
            ev = jnp.exp(m_v[sl] - mt)
            plsc.addupdate_scatter(w_v, [si], ev)
        return carry

    lax.fori_loop(0, NITER // UNROLL, body2, 0)

    mrow_v[...] = jnp.full((LANES,), mt, dtype=jnp.float32)
    for b in range(NB):
        pltpu.sync_copy(w_v.at[pl.ds(b * BN, BN)], w_out.at[b, wid])
    pltpu.sync_copy(mrow_v, m_out.at[wid])


def _sc_call(ei_flat, t, p1, p2):
    mesh = plsc.VectorSubcoreMesh(core_axis_name="c", subcore_axis_name="s")
    f = functools.partial(
        pl.kernel,
        mesh=mesh,
        compiler_params=pltpu.CompilerParams(
            needs_layout_passes=False, use_tc_tiling_on_sc=False),
        out_type=[
            jax.ShapeDtypeStruct((NB, NW, BN), jnp.float32),
            jax.ShapeDtypeStruct((NW, LANES), jnp.float32),
        ],
        scratch_types=[
            pltpu.VMEM((EPW,), jnp.int32),
            pltpu.VMEM((EPW,), jnp.int32),
            pltpu.VMEM((EPW,), jnp.float32),
            pltpu.VMEM((N,), jnp.float32),
            pltpu.VMEM((N,), jnp.float32),
            pltpu.VMEM((EPW,), jnp.float32),
            pltpu.VMEM((N,), jnp.float32),
            pltpu.VMEM((LANES,), jnp.float32),
        ],
    )(_sc_edge_body)
    return f(ei_flat, t, p1, p2)


def _final_body(w_ref, mloc_ref, g_ref, w1, b1, w2, b2, w3, b3, w4, b4,
                out_ref, acc_ref, accs_ref):
    i = pl.program_id(0)

    @pl.when(i == 0)
    def _init():
        acc_ref[...] = jnp.zeros_like(acc_ref)
        accs_ref[0, 0] = 0.0

    mloc = mloc_ref[...]
    gmax = jnp.max(mloc)
    c = jnp.exp(mloc[:, 0:1] - gmax)
    wblk = w_ref[0]
    cw = jnp.sum(wblk * c, axis=0, keepdims=True)
    acc_ref[...] += jnp.dot(cw, g_ref[...], preferred_element_type=jnp.float32)
    accs_ref[0, 0] += jnp.sum(cw)

    @pl.when(i == pl.num_programs(0) - 1)
    def _finish():
        hg = acc_ref[...] / accs_ref[0, 0]
        o = jnp.dot(hg, w1[...], preferred_element_type=jnp.float32) + b1[...]
        o = jnp.maximum(o, 0.0)
        o = jnp.dot(o, w2[...], preferred_element_type=jnp.float32) + b2[...]
        o = jnp.maximum(o, 0.0)
        o = jnp.dot(o, w3[...], preferred_element_type=jnp.float32) + b3[...]
        o = jnp.maximum(o, 0.0)
        out_ref[...] = (jnp.dot(o, w4[...], preferred_element_type=jnp.float32)
                        + b4[...])


def _final_call(w5, Mloc, g, W1, b1, W2, b2, W3, b3, W4, b4):
    full = lambda i: (0, 0)
    return pl.pallas_call(
        _final_body,
        grid=(NB,),
        in_specs=[
            pl.BlockSpec((1, NW, BN), lambda i: (i, 0, 0)),
            pl.BlockSpec((NW, LANES), full),
            pl.BlockSpec((BN, H), lambda i: (i, 0)),
            pl.BlockSpec(W1.shape, full),
            pl.BlockSpec(b1.shape, full),
            pl.BlockSpec(W2.shape, full),
            pl.BlockSpec(b2.shape, full),
            pl.BlockSpec(W3.shape, full),
            pl.BlockSpec(b3.shape, full),
            pl.BlockSpec(W4.shape, full),
            pl.BlockSpec(b4.shape, full),
        ],
        out_specs=pl.BlockSpec((1, 1), full),
        out_shape=jax.ShapeDtypeStruct((1, 1), jnp.float32),
        scratch_shapes=[
            pltpu.VMEM((1, H), jnp.float32),
            pltpu.SMEM((1, 1), jnp.float32),
        ],
    )(w5, Mloc, g, W1, b1, W2, b2, W3, b3, W4, b4)


def kernel(x, edge_index, edge_attr, W_u, a, W_e, W_m,
           W1, b1, W2, b2, W3, b3, W4, b4):
    A2 = jnp.concatenate([a[:H], a[H:]], axis=1)

    g, p1_3, p2_3 = _a_call(x, W_u, A2)
    t2d = _b_call(edge_attr, W_e, W_m)

    w5, Mloc = _sc_call(edge_index.reshape(2 * E), t2d.reshape(E),
                        p1_3.reshape(N), p2_3.reshape(N))

    return _final_call(w5, Mloc, g,
                       W1, b1.reshape(1, -1), W2, b2.reshape(1, -1),
                       W3, b3.reshape(1, -1), W4, b4.reshape(1, -1))


# speedup vs baseline: 1.8965x; 1.8965x over previous
import functools

import jax
import jax.numpy as jnp
from jax import lax
from jax.experimental import pallas as pl
from jax.experimental.pallas import tpu as pltpu
from jax.experimental.pallas import tpu_sc as plsc

N = 10000
E = 320000
H = 128
NC = 2
NS = 16
NW = NC * NS
EPW = E // NW
LANES = 16
BN = 2000
NB = N // BN
BE = 6400
GB = E // BE
UNROLL = 5
NITER = EPW // LANES


def _a_body(x_ref, wu_ref, a2_ref, g_ref, p1_ref, p2_ref):
    u = jnp.dot(x_ref[...], wu_ref[...], preferred_element_type=jnp.float32)
    hp = jnp.where(u > 0, u, 0.01 * u)
    g_ref[...] = jnp.where(u > 0, u, 0.0001 * u)
    p2v = lax.dot_general(a2_ref[...], hp, (((0,), (1,)), ((), ())),
                          preferred_element_type=jnp.float32)
    p1_ref[...] = p2v[0:1, :].reshape(1, 1, BN)
    p2_ref[...] = p2v[1:2, :].reshape(1, 1, BN)


def _a_call(x, W_u, A2):
    return pl.pallas_call(
        _a_body,
        grid=(NB,),
        in_specs=[
            pl.BlockSpec((BN, H), lambda i: (i, 0)),
            pl.BlockSpec((H, H), lambda i: (0, 0)),
            pl.BlockSpec((H, 2), lambda i: (0, 0)),
        ],
        out_specs=[
            pl.BlockSpec((BN, H), lambda i: (i, 0)),
            pl.BlockSpec((1, 1, BN), lambda i: (i, 0, 0)),
            pl.BlockSpec((1, 1, BN), lambda i: (i, 0, 0)),
        ],
        out_shape=[
            jax.ShapeDtypeStruct((N, H), jnp.float32),
            jax.ShapeDtypeStruct((NB, 1, BN), jnp.float32),
            jax.ShapeDtypeStruct((NB, 1, BN), jnp.float32),
        ],
    )(x, W_u, A2)


def _b_body(eat_ref, we_ref, wm_ref, t_ref, scr_ref, sem):
    i = pl.program_id(0)
    wem = jnp.dot(we_ref[...], wm_ref[...], preferred_element_type=jnp.float32)
    sT = lax.dot_general(wem, eat_ref[...], (((0,), (0,)), ((), ())),
                         preferred_element_type=jnp.float32)
    scr_ref[...] = sT
    cp = pltpu.make_async_copy(scr_ref, t_ref.at[pl.ds(i, 1), :], sem)
    cp.start()
    cp.wait()


def _b_call(eaT, W_e, W_m):
    d_edge = W_e.shape[0]
    return pl.pallas_call(
        _b_body,
        grid=(GB,),
        in_specs=[
            pl.BlockSpec((d_edge, BE), lambda i: (0, i)),
            pl.BlockSpec((d_edge, H), lambda i: (0, 0)),
            pl.BlockSpec((H, 1), lambda i: (0, 0)),
        ],
        out_specs=pl.BlockSpec(memory_space=pltpu.MemorySpace.HBM),
        out_shape=jax.ShapeDtypeStruct((GB, BE), jnp.float32),
        scratch_shapes=[pltpu.VMEM((1, BE), jnp.float32),
                        pltpu.SemaphoreType.DMA],
    )(eaT, W_e, W_m)


def _sc_edge_body(ei_hbm, t_hbm, p1_hbm, p2_hbm, w_out, m_out,
                  src_v, dst_v, t_v, p1_v, p2_v, m_v, w_v, mrow_v):
    wid = lax.axis_index("s") * NC + lax.axis_index("c")
    base = wid * EPW
    pltpu.sync_copy(ei_hbm.at[pl.ds(base, EPW)], src_v)
    pltpu.sync_copy(ei_hbm.at[pl.ds(E + base, EPW)], dst_v)
    pltpu.sync_copy(t_hbm.at[pl.ds(base, EPW)], t_v)
    pltpu.sync_copy(p1_hbm, p1_v)
    pltpu.sync_copy(p2_hbm, p2_v)

    def body1(i, mx):
        for j in range(UNROLL):
            sl = pl.ds(pl.multiple_of((i * UNROLL + j) * LANES, LANES), LANES)
            si = src_v[sl]
            di = dst_v[sl]
            sc = plsc.load_gather(p1_v, [si]) + plsc.load_gather(p2_v, [di])
            e = jnp.exp(-jnp.abs(sc))
            sig = jnp.where(sc >= 0, 1.0 / (1.0 + e), e / (1.0 + e))
            m = sig * t_v[sl]
            m_v[sl] = m
            w_v[sl] = jnp.zeros((LANES,), jnp.float32)
            mx = jnp.maximum(mx, m)
        return mx

    mx0 = jnp.full((LANES,), -jnp.inf, dtype=jnp.float32)
    mx = lax.fori_loop(0, NITER // UNROLL, body1, mx0)
    mt = jnp.max(mx)

    def body2(i, carry):
        for j in range(UNROLL):
            sl = pl.ds(pl.multiple_of((i * UNROLL + j) * LANES, LANES), LANES)
            si = src_v[sl]
            ev = jnp.exp(m_v[sl] - mt)
            plsc.addupdate_scatter(w_v, [si], ev)
        return carry

    lax.fori_loop(0, NITER // UNROLL, body2, 0)

    mrow_v[...] = jnp.full((LANES,), mt, dtype=jnp.float32)
    for b in range(NB):
        pltpu.sync_copy(w_v.at[pl.ds(b * BN, BN)], w_out.at[b, wid])
    pltpu.sync_copy(mrow_v, m_out.at[wid])


def _sc_call(ei_flat, t, p1, p2):
    mesh = plsc.VectorSubcoreMesh(core_axis_name="c", subcore_axis_name="s")
    f = functools.partial(
        pl.kernel,
        mesh=mesh,
        compiler_params=pltpu.CompilerParams(
            needs_layout_passes=False, use_tc_tiling_on_sc=False),
        out_type=[
            jax.ShapeDtypeStruct((NB, NW, BN), jnp.float32),
            jax.ShapeDtypeStruct((NW, LANES), jnp.float32),
        ],
        scratch_types=[
            pltpu.VMEM((EPW,), jnp.int32),
            pltpu.VMEM((EPW,), jnp.int32),
            pltpu.VMEM((EPW,), jnp.float32),
            pltpu.VMEM((N,), jnp.float32),
            pltpu.VMEM((N,), jnp.float32),
            pltpu.VMEM((EPW,), jnp.float32),
            pltpu.VMEM((N,), jnp.float32),
            pltpu.VMEM((LANES,), jnp.float32),
        ],
    )(_sc_edge_body)
    return f(ei_flat, t, p1, p2)


def _final_body(w_ref, mloc_ref, g_ref, w1, b1, w2, b2, w3, b3, w4, b4,
                out_ref, acc_ref, accs_ref):
    i = pl.program_id(0)

    @pl.when(i == 0)
    def _init():
        acc_ref[...] = jnp.zeros_like(acc_ref)
        accs_ref[0, 0] = 0.0

    mloc = mloc_ref[...]
    gmax = jnp.max(mloc)
    c = jnp.exp(mloc[:, 0:1] - gmax)
    wblk = w_ref[0]
    cw = jnp.sum(wblk * c, axis=0, keepdims=True)
    acc_ref[...] += jnp.dot(cw, g_ref[...], preferred_element_type=jnp.float32)
    accs_ref[0, 0] += jnp.sum(cw)

    @pl.when(i == pl.num_programs(0) - 1)
    def _finish():
        hg = acc_ref[...] / accs_ref[0, 0]
        o = jnp.dot(hg, w1[...], preferred_element_type=jnp.float32) + b1[...]
        o = jnp.maximum(o, 0.0)
        o = jnp.dot(o, w2[...], preferred_element_type=jnp.float32) + b2[...]
        o = jnp.maximum(o, 0.0)
        o = jnp.dot(o, w3[...], preferred_element_type=jnp.float32) + b3[...]
        o = jnp.maximum(o, 0.0)
        out_ref[...] = (jnp.dot(o, w4[...], preferred_element_type=jnp.float32)
                        + b4[...])


def _final_call(w5, Mloc, g, W1, b1, W2, b2, W3, b3, W4, b4):
    full = lambda i: (0, 0)
    return pl.pallas_call(
        _final_body,
        grid=(NB,),
        in_specs=[
            pl.BlockSpec((1, NW, BN), lambda i: (i, 0, 0)),
            pl.BlockSpec((NW, LANES), full),
            pl.BlockSpec((BN, H), lambda i: (i, 0)),
            pl.BlockSpec(W1.shape, full),
            pl.BlockSpec(b1.shape, full),
            pl.BlockSpec(W2.shape, full),
            pl.BlockSpec(b2.shape, full),
            pl.BlockSpec(W3.shape, full),
            pl.BlockSpec(b3.shape, full),
            pl.BlockSpec(W4.shape, full),
            pl.BlockSpec(b4.shape, full),
        ],
        out_specs=pl.BlockSpec((1, 1), full),
        out_shape=jax.ShapeDtypeStruct((1, 1), jnp.float32),
        scratch_shapes=[
            pltpu.VMEM((1, H), jnp.float32),
            pltpu.SMEM((1, 1), jnp.float32),
        ],
    )(w5, Mloc, g, W1, b1, W2, b2, W3, b3, W4, b4)


def kernel(x, edge_index, edge_attr, W_u, a, W_e, W_m,
           W1, b1, W2, b2, W3, b3, W4, b4):
    A2 = jnp.concatenate([a[:H], a[H:]], axis=1)

    g, p1_3, p2_3 = _a_call(x, W_u, A2)
    t2d = _b_call(edge_attr.T, W_e, W_m)

    w5, Mloc = _sc_call(edge_index.reshape(2 * E), t2d.reshape(E),
                        p1_3.reshape(N), p2_3.reshape(N))

    return _final_call(w5, Mloc, g,
                       W1, b1.reshape(1, -1), W2, b2.reshape(1, -1),
                       W3, b3.reshape(1, -1), W4, b4.reshape(1, -1))


# speedup vs baseline: 2.6524x; 1.3985x over previous
import functools

import jax
import jax.numpy as jnp
from jax import lax
from jax.experimental import pallas as pl
from jax.experimental.pallas import tpu as pltpu
from jax.experimental.pallas import tpu_sc as plsc

N = 10000
E = 320000
H = 128
NC = 2
NS = 16
NW = NC * NS
EPW = E // NW
LANES = 16
BN = 2000
NB = N // BN
BE = 64000
GB = E // BE
UNROLL = 5
NITER = EPW // LANES


def _a_body(x_ref, wu_ref, a2_ref, g_ref, p1_ref, p2_ref):
    u = jnp.dot(x_ref[...], wu_ref[...], preferred_element_type=jnp.float32)
    hp = jnp.where(u > 0, u, 0.01 * u)
    g_ref[...] = jnp.where(u > 0, u, 0.0001 * u)
    p2v = lax.dot_general(a2_ref[...], hp, (((0,), (1,)), ((), ())),
                          preferred_element_type=jnp.float32)
    p1_ref[...] = p2v[0:1, :].reshape(1, 1, BN)
    p2_ref[...] = p2v[1:2, :].reshape(1, 1, BN)


def _a_call(x, W_u, A2):
    return pl.pallas_call(
        _a_body,
        grid=(NB,),
        in_specs=[
            pl.BlockSpec((BN, H), lambda i: (i, 0)),
            pl.BlockSpec((H, H), lambda i: (0, 0)),
            pl.BlockSpec((H, 2), lambda i: (0, 0)),
        ],
        out_specs=[
            pl.BlockSpec((BN, H), lambda i: (i, 0)),
            pl.BlockSpec((1, 1, BN), lambda i: (i, 0, 0)),
            pl.BlockSpec((1, 1, BN), lambda i: (i, 0, 0)),
        ],
        out_shape=[
            jax.ShapeDtypeStruct((N, H), jnp.float32),
            jax.ShapeDtypeStruct((NB, 1, BN), jnp.float32),
            jax.ShapeDtypeStruct((NB, 1, BN), jnp.float32),
        ],
    )(x, W_u, A2)


def _b_body(eat_ref, we_ref, wm_ref, t_ref, tm_ref, scr_ref, sem, tmax_ref):
    i = pl.program_id(0)
    wem = jnp.dot(we_ref[...], wm_ref[...], preferred_element_type=jnp.float32)
    sT = lax.dot_general(wem, eat_ref[...], (((0,), (0,)), ((), ())),
                         preferred_element_type=jnp.float32)
    scr_ref[...] = sT
    bm = jnp.max(jnp.abs(sT))

    @pl.when(i == 0)
    def _():
        tmax_ref[0, 0] = bm

    @pl.when(i > 0)
    def _():
        tmax_ref[0, 0] = jnp.maximum(tmax_ref[0, 0], bm)

    cp = pltpu.make_async_copy(scr_ref, t_ref.at[pl.ds(i, 1), :], sem)
    cp.start()
    cp.wait()

    @pl.when(i == pl.num_programs(0) - 1)
    def _():
        tm_ref[...] = jnp.full((1, H), tmax_ref[0, 0], dtype=jnp.float32)


def _b_call(eaT, W_e, W_m):
    d_edge = W_e.shape[0]
    return pl.pallas_call(
        _b_body,
        grid=(GB,),
        in_specs=[
            pl.BlockSpec((d_edge, BE), lambda i: (0, i)),
            pl.BlockSpec((d_edge, H), lambda i: (0, 0)),
            pl.BlockSpec((H, 1), lambda i: (0, 0)),
        ],
        out_specs=[
            pl.BlockSpec(memory_space=pltpu.MemorySpace.HBM),
            pl.BlockSpec((1, H), lambda i: (0, 0)),
        ],
        out_shape=[
            jax.ShapeDtypeStruct((GB, BE), jnp.float32),
            jax.ShapeDtypeStruct((1, H), jnp.float32),
        ],
        scratch_shapes=[pltpu.VMEM((1, BE), jnp.float32),
                        pltpu.SemaphoreType.DMA,
                        pltpu.SMEM((1, 1), jnp.float32)],
    )(eaT, W_e, W_m)


def _sc_edge_body(ei_hbm, t_hbm, p1_hbm, p2_hbm, tm_hbm, w_out,
                  src_v, dst_v, t_v, p1_v, p2_v, w_v, tm_v):
    wid = lax.axis_index("s") * NC + lax.axis_index("c")
    base = wid * EPW
    pltpu.sync_copy(ei_hbm.at[pl.ds(base, EPW)], src_v)
    pltpu.sync_copy(ei_hbm.at[pl.ds(E + base, EPW)], dst_v)
    pltpu.sync_copy(t_hbm.at[pl.ds(base, EPW)], t_v)
    pltpu.sync_copy(p1_hbm, p1_v)
    pltpu.sync_copy(p2_hbm, p2_v)
    pltpu.sync_copy(tm_hbm, tm_v)
    cc = jnp.max(tm_v[pl.ds(0, LANES)])

    def bzero(i, carry):
        w_v[pl.ds(pl.multiple_of(i * LANES, LANES), LANES)] = (
            jnp.zeros((LANES,), jnp.float32))
        return carry

    lax.fori_loop(0, N // LANES, bzero, 0)

    def body(i, carry):
        for j in range(UNROLL):
            sl = pl.ds(pl.multiple_of((i * UNROLL + j) * LANES, LANES), LANES)
            si = src_v[sl]
            di = dst_v[sl]
            sc = plsc.load_gather(p1_v, [si]) + plsc.load_gather(p2_v, [di])
            e = jnp.exp(-jnp.abs(sc))
            r = 1.0 / (1.0 + e)
            sig = jnp.where(sc >= 0, r, 1.0 - r)
            ev = jnp.exp(sig * t_v[sl] - cc)
            plsc.addupdate_scatter(w_v, [si], ev)
        return carry

    lax.fori_loop(0, NITER // UNROLL, body, 0)

    for b in range(NB):
        pltpu.sync_copy(w_v.at[pl.ds(b * BN, BN)], w_out.at[b, wid])


def _sc_call(ei_flat, t, p1, p2, tm):
    mesh = plsc.VectorSubcoreMesh(core_axis_name="c", subcore_axis_name="s")
    f = functools.partial(
        pl.kernel,
        mesh=mesh,
        compiler_params=pltpu.CompilerParams(
            needs_layout_passes=False, use_tc_tiling_on_sc=False),
        out_type=jax.ShapeDtypeStruct((NB, NW, BN), jnp.float32),
        scratch_types=[
            pltpu.VMEM((EPW,), jnp.int32),
            pltpu.VMEM((EPW,), jnp.int32),
            pltpu.VMEM((EPW,), jnp.float32),
            pltpu.VMEM((N,), jnp.float32),
            pltpu.VMEM((N,), jnp.float32),
            pltpu.VMEM((N,), jnp.float32),
            pltpu.VMEM((H,), jnp.float32),
        ],
    )(_sc_edge_body)
    return f(ei_flat, t, p1, p2, tm)


def _final_body(w_ref, g_ref, w1, b1, w2, b2, w3, b3, w4, b4,
                out_ref, acc_ref, accs_ref):
    i = pl.program_id(0)

    @pl.when(i == 0)
    def _init():
        acc_ref[...] = jnp.zeros_like(acc_ref)
        accs_ref[0, 0] = 0.0

    wblk = w_ref[0]
    cw = jnp.sum(wblk, axis=0, keepdims=True)
    acc_ref[...] += jnp.dot(cw, g_ref[...], preferred_element_type=jnp.float32)
    accs_ref[0, 0] += jnp.sum(cw)

    @pl.when(i == pl.num_programs(0) - 1)
    def _finish():
        hg = acc_ref[...] / accs_ref[0, 0]
        o = jnp.dot(hg, w1[...], preferred_element_type=jnp.float32) + b1[...]
        o = jnp.maximum(o, 0.0)
        o = jnp.dot(o, w2[...], preferred_element_type=jnp.float32) + b2[...]
        o = jnp.maximum(o, 0.0)
        o = jnp.dot(o, w3[...], preferred_element_type=jnp.float32) + b3[...]
        o = jnp.maximum(o, 0.0)
        out_ref[...] = (jnp.dot(o, w4[...], preferred_element_type=jnp.float32)
                        + b4[...])


def _final_call(w5, g, W1, b1, W2, b2, W3, b3, W4, b4):
    full = lambda i: (0, 0)
    return pl.pallas_call(
        _final_body,
        grid=(NB,),
        in_specs=[
            pl.BlockSpec((1, NW, BN), lambda i: (i, 0, 0)),
            pl.BlockSpec((BN, H), lambda i: (i, 0)),
            pl.BlockSpec(W1.shape, full),
            pl.BlockSpec(b1.shape, full),
            pl.BlockSpec(W2.shape, full),
            pl.BlockSpec(b2.shape, full),
            pl.BlockSpec(W3.shape, full),
            pl.BlockSpec(b3.shape, full),
            pl.BlockSpec(W4.shape, full),
            pl.BlockSpec(b4.shape, full),
        ],
        out_specs=pl.BlockSpec((1, 1), full),
        out_shape=jax.ShapeDtypeStruct((1, 1), jnp.float32),
        scratch_shapes=[
            pltpu.VMEM((1, H), jnp.float32),
            pltpu.SMEM((1, 1), jnp.float32),
        ],
    )(w5, g, W1, b1, W2, b2, W3, b3, W4, b4)


def kernel(x, edge_index, edge_attr, W_u, a, W_e, W_m,
           W1, b1, W2, b2, W3, b3, W4, b4):
    A2 = jnp.concatenate([a[:H], a[H:]], axis=1)

    g, p1_3, p2_3 = _a_call(x, W_u, A2)
    t2d, tm = _b_call(edge_attr.T, W_e, W_m)

    w5 = _sc_call(edge_index.reshape(2 * E), t2d.reshape(E),
                  p1_3.reshape(N), p2_3.reshape(N), tm.reshape(H))

    return _final_call(w5, g,
                       W1, b1.reshape(1, -1), W2, b2.reshape(1, -1),
                       W3, b3.reshape(1, -1), W4, b4.reshape(1, -1))


# speedup vs baseline: 3.3662x; 1.2692x over previous
import functools

import jax
import jax.numpy as jnp
from jax import lax
from jax.experimental import pallas as pl
from jax.experimental.pallas import tpu as pltpu
from jax.experimental.pallas import tpu_sc as plsc

N = 10000
E = 320000
H = 128
NC = 2
NS = 16
NW = NC * NS
EPW = E // NW
LANES = 16
BN = 2000
NB = N // BN
BE = 80000
GB = E // BE
TPR = BE // EPW
UNROLL = 5
NITER = EPW // LANES


def _a_body(x_ref, wu_ref, a2_ref, g_ref, p_ref, scr_ref, sem):
    i = pl.program_id(0)
    u = jnp.dot(x_ref[...], wu_ref[...], preferred_element_type=jnp.float32)
    hp = jnp.where(u > 0, u, 0.01 * u)
    g_ref[...] = jnp.where(u > 0, u, 0.0001 * u)
    scr_ref[...] = lax.dot_general(a2_ref[...], hp, (((0,), (1,)), ((), ())),
                                   preferred_element_type=jnp.float32)
    cp = pltpu.make_async_copy(scr_ref, p_ref.at[i], sem)
    cp.start()
    cp.wait()


def _a_call(x, W_u, A2):
    return pl.pallas_call(
        _a_body,
        grid=(NB,),
        in_specs=[
            pl.BlockSpec((BN, H), lambda i: (i, 0)),
            pl.BlockSpec((H, H), lambda i: (0, 0)),
            pl.BlockSpec((H, 2), lambda i: (0, 0)),
        ],
        out_specs=[
            pl.BlockSpec((BN, H), lambda i: (i, 0)),
            pl.BlockSpec(memory_space=pltpu.MemorySpace.HBM),
        ],
        out_shape=[
            jax.ShapeDtypeStruct((N, H), jnp.float32),
            jax.ShapeDtypeStruct((NB, 2, BN), jnp.float32),
        ],
        scratch_shapes=[pltpu.VMEM((2, BN), jnp.float32),
                        pltpu.SemaphoreType.DMA],
    )(x, W_u, A2)


def _b_body(eat_ref, we_ref, wm_ref, t_ref, tm_ref, scr_ref, sem, tmax_ref):
    i = pl.program_id(0)
    wem = jnp.dot(we_ref[...], wm_ref[...], preferred_element_type=jnp.float32)
    sT = lax.dot_general(wem, eat_ref[...], (((0,), (0,)), ((), ())),
                         preferred_element_type=jnp.float32)
    scr_ref[...] = sT
    bm = jnp.max(jnp.abs(sT))

    @pl.when(i == 0)
    def _():
        tmax_ref[0, 0] = bm

    @pl.when(i > 0)
    def _():
        tmax_ref[0, 0] = jnp.maximum(tmax_ref[0, 0], bm)

    cp = pltpu.make_async_copy(scr_ref, t_ref.at[pl.ds(i, 1), :], sem)
    cp.start()
    cp.wait()

    @pl.when(i == pl.num_programs(0) - 1)
    def _():
        tm_ref[...] = jnp.full((1, H), tmax_ref[0, 0], dtype=jnp.float32)


def _b_call(eaT, W_e, W_m):
    d_edge = W_e.shape[0]
    return pl.pallas_call(
        _b_body,
        grid=(GB,),
        in_specs=[
            pl.BlockSpec((d_edge, BE), lambda i: (0, i)),
            pl.BlockSpec((d_edge, H), lambda i: (0, 0)),
            pl.BlockSpec((H, 1), lambda i: (0, 0)),
        ],
        out_specs=[
            pl.BlockSpec(memory_space=pltpu.MemorySpace.HBM),
            pl.BlockSpec((1, H), lambda i: (0, 0)),
        ],
        out_shape=[
            jax.ShapeDtypeStruct((GB, BE), jnp.float32),
            jax.ShapeDtypeStruct((1, H), jnp.float32),
        ],
        scratch_shapes=[pltpu.VMEM((1, BE), jnp.float32),
                        pltpu.SemaphoreType.DMA,
                        pltpu.SMEM((1, 1), jnp.float32)],
    )(eaT, W_e, W_m)


def _sc_edge_body(ei_hbm, t_hbm, p_hbm, tm_hbm, w_out,
                  src_v, dst_v, t_v, p1_v, p2_v, w_v, tm_v, sem):
    wid = lax.axis_index("s") * NC + lax.axis_index("c")
    base = wid * EPW
    cps = [
        pltpu.make_async_copy(ei_hbm.at[pl.ds(base, EPW)], src_v, sem),
        pltpu.make_async_copy(ei_hbm.at[pl.ds(E + base, EPW)], dst_v, sem),
        pltpu.make_async_copy(
            t_hbm.at[wid // TPR, pl.ds((wid % TPR) * EPW, EPW)], t_v, sem),
        pltpu.make_async_copy(tm_hbm, tm_v, sem),
    ] + [
        pltpu.make_async_copy(p_hbm.at[b, 0], p1_v.at[pl.ds(b * BN, BN)], sem)
        for b in range(NB)
    ] + [
        pltpu.make_async_copy(p_hbm.at[b, 1], p2_v.at[pl.ds(b * BN, BN)], sem)
        for b in range(NB)
    ]
    for cp in cps:
        cp.start()

    def bzero(i, carry):
        w_v[pl.ds(pl.multiple_of(i * LANES, LANES), LANES)] = (
            jnp.zeros((LANES,), jnp.float32))
        return carry

    lax.fori_loop(0, N // LANES, bzero, 0)
    for cp in cps:
        cp.wait()
    cc = jnp.max(tm_v[pl.ds(0, LANES)])

    def body(i, carry):
        sls, sis, scs = [], [], []
        for j in range(UNROLL):
            sl = pl.ds(pl.multiple_of((i * UNROLL + j) * LANES, LANES), LANES)
            si = src_v[sl]
            di = dst_v[sl]
            sc = plsc.load_gather(p1_v, [si]) + plsc.load_gather(p2_v, [di])
            sls.append(sl)
            sis.append(si)
            scs.append(sc)
        es = [jnp.exp(-jnp.abs(sc)) for sc in scs]
        zs = []
        for j in range(UNROLL):
            d = 1.0 + es[j]
            r = 1.4117647 - 0.4705882 * d
            r = r * (2.0 - d * r)
            r = r * (2.0 - d * r)
            sig = jnp.where(scs[j] >= 0, r, 1.0 - r)
            zs.append(sig * t_v[sls[j]] - cc)
        exs = [jnp.exp(z) for z in zs]
        for j in range(UNROLL):
            plsc.addupdate_scatter(w_v, [sis[j]], exs[j])
        return carry

    lax.fori_loop(0, NITER // UNROLL, body, 0)

    for b in range(NB):
        pltpu.sync_copy(w_v.at[pl.ds(b * BN, BN)], w_out.at[b, wid])


def _sc_call(ei_flat, t2d, p, tm):
    mesh = plsc.VectorSubcoreMesh(core_axis_name="c", subcore_axis_name="s")
    f = functools.partial(
        pl.kernel,
        mesh=mesh,
        compiler_params=pltpu.CompilerParams(
            needs_layout_passes=False, use_tc_tiling_on_sc=False),
        out_type=jax.ShapeDtypeStruct((NB, NW, BN), jnp.float32),
        scratch_types=[
            pltpu.VMEM((EPW,), jnp.int32),
            pltpu.VMEM((EPW,), jnp.int32),
            pltpu.VMEM((EPW,), jnp.float32),
            pltpu.VMEM((N,), jnp.float32),
            pltpu.VMEM((N,), jnp.float32),
            pltpu.VMEM((N,), jnp.float32),
            pltpu.VMEM((H,), jnp.float32),
            pltpu.SemaphoreType.DMA,
        ],
    )(_sc_edge_body)
    return f(ei_flat, t2d, p, tm)


def _final_body(w_ref, g_ref, w1, b1, w2, b2, w3, b3, w4, b4,
                out_ref, wscr_ref, sem, acc_ref, accs_ref):
    i = pl.program_id(0)

    @pl.when(i == 0)
    def _init():
        acc_ref[...] = jnp.zeros_like(acc_ref)
        accs_ref[0, 0] = 0.0

    cp = pltpu.make_async_copy(w_ref.at[i], wscr_ref, sem)
    cp.start()
    cp.wait()
    wblk = wscr_ref[...]
    cw = jnp.sum(wblk, axis=0, keepdims=True)
    acc_ref[...] += jnp.dot(cw, g_ref[...], preferred_element_type=jnp.float32)
    accs_ref[0, 0] += jnp.sum(cw)

    @pl.when(i == pl.num_programs(0) - 1)
    def _finish():
        hg = acc_ref[...] / accs_ref[0, 0]
        o = jnp.dot(hg, w1[...], preferred_element_type=jnp.float32) + b1[...]
        o = jnp.maximum(o, 0.0)
        o = jnp.dot(o, w2[...], preferred_element_type=jnp.float32) + b2[...]
        o = jnp.maximum(o, 0.0)
        o = jnp.dot(o, w3[...], preferred_element_type=jnp.float32) + b3[...]
        o = jnp.maximum(o, 0.0)
        out_ref[...] = (jnp.dot(o, w4[...], preferred_element_type=jnp.float32)
                        + b4[...])


def _final_call(w5, g, W1, b1, W2, b2, W3, b3, W4, b4):
    full = lambda i: (0, 0)
    return pl.pallas_call(
        _final_body,
        grid=(NB,),
        in_specs=[
            pl.BlockSpec(memory_space=pltpu.MemorySpace.HBM),
            pl.BlockSpec((BN, H), lambda i: (i, 0)),
            pl.BlockSpec(W1.shape, full),
            pl.BlockSpec(b1.shape, full),
            pl.BlockSpec(W2.shape, full),
            pl.BlockSpec(b2.shape, full),
            pl.BlockSpec(W3.shape, full),
            pl.BlockSpec(b3.shape, full),
            pl.BlockSpec(W4.shape, full),
            pl.BlockSpec(b4.shape, full),
        ],
        out_specs=pl.BlockSpec((1, 1), full),
        out_shape=jax.ShapeDtypeStruct((1, 1), jnp.float32),
        scratch_shapes=[
            pltpu.VMEM((NW, BN), jnp.float32),
            pltpu.SemaphoreType.DMA,
            pltpu.VMEM((1, H), jnp.float32),
            pltpu.SMEM((1, 1), jnp.float32),
        ],
    )(w5, g, W1, b1, W2, b2, W3, b3, W4, b4)


def kernel(x, edge_index, edge_attr, W_u, a, W_e, W_m,
           W1, b1, W2, b2, W3, b3, W4, b4):
    A2 = jnp.concatenate([a[:H], a[H:]], axis=1)

    g, p = _a_call(x, W_u, A2)
    t2d, tm = _b_call(edge_attr.T, W_e, W_m)

    w5 = _sc_call(edge_index.reshape(2 * E), t2d, p, tm.reshape(H))

    return _final_call(w5, g,
                       W1, b1.reshape(1, -1), W2, b2.reshape(1, -1),
                       W3, b3.reshape(1, -1), W4, b4.reshape(1, -1))


# speedup vs baseline: 4.2034x; 1.2487x over previous
import functools

import jax
import jax.numpy as jnp
from jax import lax
from jax.experimental import pallas as pl
from jax.experimental.pallas import tpu as pltpu
from jax.experimental.pallas import tpu_sc as plsc

N = 10000
E = 320000
H = 128
NC = 2
NS = 16
NW = NC * NS
EPW = E // NW
LANES = 16
BN = 2000
NB = N // BN
BE = E // NB
UNROLL = 5
NITER = EPW // LANES
NP = 2048
PN = NB * NP


def _ab_body(x_ref, wu_ref, a2_ref, eat_ref, we_ref, wm_ref, ei_ref,
             g_ref, p_ref, t_ref, tm_ref, ei_out,
             pscr_ref, tscr_ref, sem, tmax_ref):
    i = pl.program_id(0)

    u = jnp.dot(x_ref[...], wu_ref[...], preferred_element_type=jnp.float32)
    hp = jnp.where(u > 0, u, 0.01 * u)
    g_ref[...] = jnp.where(u > 0, u, 0.0001 * u)
    pscr_ref[:, pl.ds(0, BN)] = lax.dot_general(
        a2_ref[...], hp, (((0,), (1,)), ((), ())),
        preferred_element_type=jnp.float32)

    wem = jnp.dot(we_ref[...], wm_ref[...], preferred_element_type=jnp.float32)
    sT = lax.dot_general(wem, eat_ref[...], (((0,), (0,)), ((), ())),
                         preferred_element_type=jnp.float32)
    tscr_ref[...] = sT
    bm = jnp.max(jnp.abs(sT))

    @pl.when(i == 0)
    def _():
        tmax_ref[0, 0] = bm

    @pl.when(i > 0)
    def _():
        tmax_ref[0, 0] = jnp.maximum(tmax_ref[0, 0], bm)

    cps = [
        pltpu.make_async_copy(pscr_ref.at[0], p_ref.at[pl.ds(i * NP, NP)],
                              sem),
        pltpu.make_async_copy(pscr_ref.at[1], p_ref.at[pl.ds(PN + i * NP, NP)],
                              sem),
        pltpu.make_async_copy(tscr_ref.at[0], t_ref.at[pl.ds(i * BE, BE)],
                              sem),
        pltpu.make_async_copy(ei_ref.at[0], ei_out.at[pl.ds(i * BE, BE)], sem),
        pltpu.make_async_copy(ei_ref.at[1],
                              ei_out.at[pl.ds(E + i * BE, BE)], sem),
    ]
    for cp in cps:
        cp.start()
    for cp in cps:
        cp.wait()

    @pl.when(i == pl.num_programs(0) - 1)
    def _():
        tm_ref[...] = jnp.full((1, H), tmax_ref[0, 0], dtype=jnp.float32)


def _ab_call(x, W_u, A2, eaT, W_e, W_m, edge_index):
    d_edge = W_e.shape[0]
    return pl.pallas_call(
        _ab_body,
        grid=(NB,),
        in_specs=[
            pl.BlockSpec((BN, H), lambda i: (i, 0)),
            pl.BlockSpec((H, H), lambda i: (0, 0)),
            pl.BlockSpec((H, 2), lambda i: (0, 0)),
            pl.BlockSpec((d_edge, BE), lambda i: (0, i)),
            pl.BlockSpec((d_edge, H), lambda i: (0, 0)),
            pl.BlockSpec((H, 1), lambda i: (0, 0)),
            pl.BlockSpec((2, BE), lambda i: (0, i)),
        ],
        out_specs=[
            pl.BlockSpec((BN, H), lambda i: (i, 0)),
            pl.BlockSpec(memory_space=pltpu.MemorySpace.HBM),
            pl.BlockSpec(memory_space=pltpu.MemorySpace.HBM),
            pl.BlockSpec((1, H), lambda i: (0, 0)),
            pl.BlockSpec(memory_space=pltpu.MemorySpace.HBM),
        ],
        out_shape=[
            jax.ShapeDtypeStruct((N, H), jnp.float32),
            jax.ShapeDtypeStruct((2 * PN,), jnp.float32),
            jax.ShapeDtypeStruct((E,), jnp.float32),
            jax.ShapeDtypeStruct((1, H), jnp.float32),
            jax.ShapeDtypeStruct((2 * E,), jnp.int32),
        ],
        scratch_shapes=[pltpu.VMEM((2, NP), jnp.float32),
                        pltpu.VMEM((1, BE), jnp.float32),
                        pltpu.SemaphoreType.DMA,
                        pltpu.SMEM((1, 1), jnp.float32)],
    )(x, W_u, A2, eaT, W_e, W_m, edge_index)


def _sc_edge_body(ei_hbm, t_hbm, p_hbm, tm_hbm, w_out,
                  src_v, dst_v, t_v, p1_v, p2_v, w_v, tm_v, sem):
    wid = lax.axis_index("s") * NC + lax.axis_index("c")
    base = wid * EPW
    cps = [
        pltpu.make_async_copy(ei_hbm.at[pl.ds(base, EPW)], src_v, sem),
        pltpu.make_async_copy(ei_hbm.at[pl.ds(E + base, EPW)], dst_v, sem),
        pltpu.make_async_copy(t_hbm.at[pl.ds(base, EPW)], t_v, sem),
        pltpu.make_async_copy(tm_hbm, tm_v, sem),
    ] + [
        pltpu.make_async_copy(p_hbm.at[pl.ds(b * NP, BN)],
                              p1_v.at[pl.ds(b * BN, BN)], sem)
        for b in range(NB)
    ] + [
        pltpu.make_async_copy(p_hbm.at[pl.ds(PN + b * NP, BN)],
                              p2_v.at[pl.ds(b * BN, BN)], sem)
        for b in range(NB)
    ]
    for cp in cps:
        cp.start()

    def bzero(i, carry):
        w_v[pl.ds(pl.multiple_of(i * LANES, LANES), LANES)] = (
            jnp.zeros((LANES,), jnp.float32))
        return carry

    lax.fori_loop(0, N // LANES, bzero, 0)
    for cp in cps:
        cp.wait()
    cc = jnp.max(tm_v[pl.ds(0, LANES)])

    def body(i, carry):
        sls, sis, scs = [], [], []
        for j in range(UNROLL):
            sl = pl.ds(pl.multiple_of((i * UNROLL + j) * LANES, LANES), LANES)
            si = src_v[sl]
            di = dst_v[sl]
            sc = plsc.load_gather(p1_v, [si]) + plsc.load_gather(p2_v, [di])
            sls.append(sl)
            sis.append(si)
            scs.append(sc)
        es = [jnp.exp(-jnp.abs(sc)) for sc in scs]
        zs = []
        for j in range(UNROLL):
            d = 1.0 + es[j]
            r = 1.4117647 - 0.4705882 * d
            r = r * (2.0 - d * r)
            r = r * (2.0 - d * r)
            sig = jnp.where(scs[j] >= 0, r, 1.0 - r)
            zs.append(sig * t_v[sls[j]] - cc)
        exs = [jnp.exp(z) for z in zs]
        for j in range(UNROLL):
            plsc.addupdate_scatter(w_v, [sis[j]], exs[j])
        return carry

    lax.fori_loop(0, NITER // UNROLL, body, 0)

    for b in range(NB):
        pltpu.sync_copy(w_v.at[pl.ds(b * BN, BN)],
                        w_out.at[pl.ds((b * NW + wid) * NP, BN)])


def _sc_call(ei_flat, t, p, tm):
    mesh = plsc.VectorSubcoreMesh(core_axis_name="c", subcore_axis_name="s")
    f = functools.partial(
        pl.kernel,
        mesh=mesh,
        compiler_params=pltpu.CompilerParams(
            needs_layout_passes=False, use_tc_tiling_on_sc=False),
        out_type=jax.ShapeDtypeStruct((NB * NW * NP,), jnp.float32),
        scratch_types=[
            pltpu.VMEM((EPW,), jnp.int32),
            pltpu.VMEM((EPW,), jnp.int32),
            pltpu.VMEM((EPW,), jnp.float32),
            pltpu.VMEM((N,), jnp.float32),
            pltpu.VMEM((N,), jnp.float32),
            pltpu.VMEM((N,), jnp.float32),
            pltpu.VMEM((H,), jnp.float32),
            pltpu.SemaphoreType.DMA,
        ],
    )(_sc_edge_body)
    return f(ei_flat, t, p, tm)


def _final_body(w_ref, g_ref, w1, b1, w2, b2, w3, b3, w4, b4,
                out_ref, wscr_ref, sem, acc_ref, accs_ref):
    i = pl.program_id(0)

    @pl.when(i == 0)
    def _init():
        acc_ref[...] = jnp.zeros_like(acc_ref)
        accs_ref[0, 0] = 0.0

    cps = [
        pltpu.make_async_copy(
            w_ref.at[pl.ds((i * NW + t) * NP, NP)], wscr_ref.at[t], sem)
        for t in range(NW)
    ]
    for cp in cps:
        cp.start()
    for cp in cps:
        cp.wait()
    wblk = wscr_ref[:, pl.ds(0, BN)]
    cw = jnp.sum(wblk, axis=0, keepdims=True)
    acc_ref[...] += jnp.dot(cw, g_ref[...], preferred_element_type=jnp.float32)
    accs_ref[0, 0] += jnp.sum(cw)

    @pl.when(i == pl.num_programs(0) - 1)
    def _finish():
        hg = acc_ref[...] / accs_ref[0, 0]
        o = jnp.dot(hg, w1[...], preferred_element_type=jnp.float32) + b1[...]
        o = jnp.maximum(o, 0.0)
        o = jnp.dot(o, w2[...], preferred_element_type=jnp.float32) + b2[...]
        o = jnp.maximum(o, 0.0)
        o = jnp.dot(o, w3[...], preferred_element_type=jnp.float32) + b3[...]
        o = jnp.maximum(o, 0.0)
        out_ref[...] = (jnp.dot(o, w4[...], preferred_element_type=jnp.float32)
                        + b4[...])


def _final_call(w1d, g, W1, b1, W2, b2, W3, b3, W4, b4):
    full = lambda i: (0, 0)
    return pl.pallas_call(
        _final_body,
        grid=(NB,),
        in_specs=[
            pl.BlockSpec(memory_space=pltpu.MemorySpace.HBM),
            pl.BlockSpec((BN, H), lambda i: (i, 0)),
            pl.BlockSpec(W1.shape, full),
            pl.BlockSpec(b1.shape, full),
            pl.BlockSpec(W2.shape, full),
            pl.BlockSpec(b2.shape, full),
            pl.BlockSpec(W3.shape, full),
            pl.BlockSpec(b3.shape, full),
            pl.BlockSpec(W4.shape, full),
            pl.BlockSpec(b4.shape, full),
        ],
        out_specs=pl.BlockSpec((1, 1), full),
        out_shape=jax.ShapeDtypeStruct((1, 1), jnp.float32),
        scratch_shapes=[
            pltpu.VMEM((NW, NP), jnp.float32),
            pltpu.SemaphoreType.DMA,
            pltpu.VMEM((1, H), jnp.float32),
            pltpu.SMEM((1, 1), jnp.float32),
        ],
    )(w1d, g, W1, b1, W2, b2, W3, b3, W4, b4)


def kernel(x, edge_index, edge_attr, W_u, a, W_e, W_m,
           W1, b1, W2, b2, W3, b3, W4, b4):
    A2 = jnp.concatenate([a[:H], a[H:]], axis=1)

    g, p, t, tm, ei = _ab_call(x, W_u, A2, edge_attr.T, W_e, W_m, edge_index)

    w1d = _sc_call(ei, t, p, tm.reshape(H))

    return _final_call(w1d, g,
                       W1, b1.reshape(1, -1), W2, b2.reshape(1, -1),
                       W3, b3.reshape(1, -1), W4, b4.reshape(1, -1))
